# Initial kernel scaffold; baseline (speedup 1.0000x reference)
import jax
import jax.numpy as jnp
from jax.experimental import pallas as pl


def kernel(point_cloud, W, b):
    raise NotImplementedError("write your pallas kernel here")


# speedup vs baseline: 28.0765x; 28.0765x over previous
import functools

import jax
import jax.numpy as jnp
from jax import lax
from jax.experimental import pallas as pl
from jax.experimental.pallas import tpu as pltpu
from jax.experimental.pallas import tpu_sc as plsc

R = 32
P = 4
G = R // P
V = R * R * R
B = 16
C = 3
N = 131072
HIDDEN = 384
SUBL = 1024
LANE = 128
CH = 4096
NCHUNK = N // CH


def _stats_body(pc_ref, idx_ref, mean_ref, scale_ref):
    p = pl.program_id(1)

    @pl.when(p == 0)
    def _():
        for ci in range(C):
            mean_ref[ci] = jnp.sum(pc_ref[0, ci]) / N

    @pl.when(p == 1)
    def _():
        n0 = pc_ref[0, 0] - mean_ref[0]
        n1 = pc_ref[0, 1] - mean_ref[1]
        n2 = pc_ref[0, 2] - mean_ref[2]
        s = n0 * n0 + n1 * n1 + n2 * n2
        scale_ref[0] = jnp.sqrt(jnp.max(s)) * 2.0

    @pl.when(p == 2)
    def _():
        md2 = scale_ref[0]

        def quant(ci):
            nc = (pc_ref[0, ci] - mean_ref[ci]) / md2 + 0.5
            v = jnp.clip(nc * float(R), 0.0, float(R - 1))
            return v.astype(jnp.int32)

        idx_ref[0] = quant(0) * (R * R) + quant(1) * R + quant(2)


def _voxel_indices(pc4):
    return pl.pallas_call(
        _stats_body,
        grid=(B, 3),
        in_specs=[pl.BlockSpec((1, C, SUBL, LANE), lambda b, p: (b, 0, 0, 0))],
        out_specs=pl.BlockSpec((1, SUBL, LANE), lambda b, p: (b, 0, 0)),
        out_shape=jax.ShapeDtypeStruct((B, SUBL, LANE), jnp.int32),
        scratch_shapes=[
            pltpu.SMEM((C,), jnp.float32),
            pltpu.SMEM((1,), jnp.float32),
        ],
    )(pc4)


def _sc_voxelize(pc, idx):
    mesh = plsc.VectorSubcoreMesh(core_axis_name="c", subcore_axis_name="s")

    @functools.partial(
        pl.kernel,
        out_type=jax.ShapeDtypeStruct((B, C, V), jnp.float32),
        mesh=mesh,
        compiler_params=pltpu.CompilerParams(
            use_tc_tiling_on_sc=False, needs_layout_passes=False
        ),
        scratch_types=[
            pltpu.VMEM((V,), jnp.float32),
            pltpu.VMEM((V,), jnp.float32),
            pltpu.VMEM((2, CH), jnp.int32),
            pltpu.VMEM((2, CH), jnp.float32),
            pltpu.VMEM((2, CH), jnp.float32),
            pltpu.VMEM_SHARED((8, V), jnp.float32),
            pltpu.SemaphoreType.DMA,
            pltpu.SemaphoreType.DMA,
        ],
    )
    def k(pc_hbm, idx_hbm, out_hbm, g0, g1, ib, f0b, f1b, recip_sh, sem0, sem1):
        c = lax.axis_index("c")
        s = lax.axis_index("s")
        bl = s % 8
        b = c * 8 + bl
        role = s // 8
        rolef = role.astype(jnp.float32)
        one_m_r = 1.0 - rolef
        ch0 = role * 2
        ch1 = 1 + role

        def zbody(i, _):
            z = jnp.zeros((16,), jnp.float32)
            g0[pl.ds(i * 16, 16)] = z
            g1[pl.ds(i * 16, 16)] = z
            return 0

        lax.fori_loop(0, V // 16, zbody, 0)

        sems = [sem0, sem1]

        def start(kk, slot):
            off = kk * CH
            return [
                pltpu.async_copy(idx_hbm.at[b, pl.ds(off, CH)], ib.at[slot], sems[slot]),
                pltpu.async_copy(pc_hbm.at[b, ch0, pl.ds(off, CH)], f0b.at[slot], sems[slot]),
                pltpu.async_copy(pc_hbm.at[b, ch1, pl.ds(off, CH)], f1b.at[slot], sems[slot]),
            ]

        handles = [start(0, 0), start(1, 1)]
        for kk in range(NCHUNK):
            slot = kk & 1
            for h in handles[slot]:
                h.wait()

            def sbody(j, _):
                off = j * 16
                iv = ib[slot, pl.ds(off, 16)]
                v0 = f0b[slot, pl.ds(off, 16)]
                v1 = f1b[slot, pl.ds(off, 16)]
                v1 = v1 * one_m_r + rolef
                plsc.addupdate_scatter(g0, [iv], v0)
                plsc.addupdate_scatter(g1, [iv], v1)
                return 0

            lax.fori_loop(0, CH // 16, sbody, 0)
            if kk + 2 < NCHUNK:
                handles[slot] = start(kk + 2, slot)

        @pl.when(role == 1)
        def _():
            def rbody(i, _):
                sl = pl.ds(i * 16, 16)
                cnt = g1[sl]
                pos = cnt > 0.0
                g1[sl] = jnp.where(pos, 1.0 / jnp.where(pos, cnt, 1.0), 0.0)
                return 0

            lax.fori_loop(0, V // 16, rbody, 0)
            pltpu.sync_copy(g1, recip_sh.at[bl])

        plsc.subcore_barrier()

        @pl.when(role == 0)
        def _():
            for kk in range(V // CH):
                pltpu.sync_copy(recip_sh.at[bl, pl.ds(kk * CH, CH)], f0b.at[0])

                def mb(j, _):
                    sl = pl.ds(kk * CH + j * 16, 16)
                    r = f0b[0, pl.ds(j * 16, 16)]
                    g0[sl] = g0[sl] * r
                    g1[sl] = g1[sl] * r
                    return 0

                lax.fori_loop(0, CH // 16, mb, 0)
            pltpu.sync_copy(g0, out_hbm.at[b, 0])
            pltpu.sync_copy(g1, out_hbm.at[b, 1])

        @pl.when(role == 1)
        def _():
            def mb(j, _):
                sl = pl.ds(j * 16, 16)
                g0[sl] = g0[sl] * g1[sl]
                return 0

            lax.fori_loop(0, V // 16, mb, 0)
            pltpu.sync_copy(g0, out_hbm.at[b, 2])

    return k(pc, idx)


def _mm_body(x_ref, w_ref, b_ref, o_ref):
    o_ref[...] = (
        jnp.dot(x_ref[...], w_ref[...], preferred_element_type=jnp.float32)
        + b_ref[...]
    )


def _patch_matmul(pat, wt, bias):
    rows = B * G * G * G
    blk = 1024
    return pl.pallas_call(
        _mm_body,
        grid=(rows // blk,),
        in_specs=[
            pl.BlockSpec((blk, C * P * P * P), lambda i: (i, 0)),
            pl.BlockSpec((C * P * P * P, HIDDEN), lambda i: (0, 0)),
            pl.BlockSpec((1, HIDDEN), lambda i: (0, 0)),
        ],
        out_specs=pl.BlockSpec((blk, HIDDEN), lambda i: (i, 0)),
        out_shape=jax.ShapeDtypeStruct((rows, HIDDEN), jnp.float32),
    )(pat, wt, bias)


def kernel(point_cloud, W, b):
    pc4 = point_cloud.reshape(B, C, SUBL, LANE)
    idx = _voxel_indices(pc4).reshape(B, N)
    avg = _sc_voxelize(point_cloud, idx)
    pat = (
        avg.reshape(B, C, G, P, G, P, G, P)
        .transpose(0, 2, 4, 6, 1, 3, 5, 7)
        .reshape(B * G * G * G, C * P * P * P)
    )
    wt = W.reshape(HIDDEN, C * P * P * P).T
    tokens = _patch_matmul(pat, wt, b.reshape(1, HIDDEN))
    return tokens.reshape(B, G * G * G, HIDDEN)


# speedup vs baseline: 34.0153x; 1.2115x over previous
import functools

import jax
import jax.numpy as jnp
from jax import lax
from jax.experimental import pallas as pl
from jax.experimental.pallas import tpu as pltpu
from jax.experimental.pallas import tpu_sc as plsc

R = 32
P = 4
G = R // P
V = R * R * R
B = 16
C = 3
N = 131072
HIDDEN = 384
SUBL = 1024
LANE = 128
CH = 4096
NCHUNK = N // CH


def _stats_body(pc_ref, idx_ref, mean_ref, scale_ref):
    p = pl.program_id(1)

    @pl.when(p == 0)
    def _():
        for ci in range(C):
            mean_ref[ci] = jnp.sum(pc_ref[0, ci]) / N

    @pl.when(p == 1)
    def _():
        n0 = pc_ref[0, 0] - mean_ref[0]
        n1 = pc_ref[0, 1] - mean_ref[1]
        n2 = pc_ref[0, 2] - mean_ref[2]
        s = n0 * n0 + n1 * n1 + n2 * n2
        scale_ref[0] = jnp.sqrt(jnp.max(s)) * 2.0

    @pl.when(p == 2)
    def _():
        md2 = scale_ref[0]

        def quant(ci):
            nc = (pc_ref[0, ci] - mean_ref[ci]) / md2 + 0.5
            v = jnp.clip(nc * float(R), 0.0, float(R - 1))
            return v.astype(jnp.int32)

        idx_ref[0] = quant(0) * (R * R) + quant(1) * R + quant(2)


def _voxel_indices(pc4):
    return pl.pallas_call(
        _stats_body,
        grid=(B, 3),
        in_specs=[pl.BlockSpec((1, C, SUBL, LANE), lambda b, p: (b, 0, 0, 0))],
        out_specs=pl.BlockSpec((1, SUBL, LANE), lambda b, p: (b, 0, 0)),
        out_shape=jax.ShapeDtypeStruct((B, SUBL, LANE), jnp.int32),
        scratch_shapes=[
            pltpu.SMEM((C,), jnp.float32),
            pltpu.SMEM((1,), jnp.float32),
        ],
    )(pc4)


ROWS = CH // LANE


def _sc_voxelize(pc4, idx4):
    mesh = plsc.VectorSubcoreMesh(core_axis_name="c", subcore_axis_name="s")

    @functools.partial(
        pl.kernel,
        out_type=jax.ShapeDtypeStruct((B, C, V), jnp.float32),
        mesh=mesh,
        compiler_params=pltpu.CompilerParams(
            use_tc_tiling_on_sc=False, needs_layout_passes=False
        ),
        scratch_types=[
            pltpu.VMEM((V,), jnp.float32),
            pltpu.VMEM((V,), jnp.float32),
            pltpu.VMEM((2, ROWS, LANE), jnp.int32),
            pltpu.VMEM((2, ROWS, LANE), jnp.float32),
            pltpu.VMEM((2, ROWS, LANE), jnp.float32),
            pltpu.VMEM((CH,), jnp.float32),
            pltpu.VMEM_SHARED((8, V), jnp.float32),
            pltpu.SemaphoreType.DMA,
            pltpu.SemaphoreType.DMA,
        ],
    )
    def k(pc_hbm, idx_hbm, out_hbm, g0, g1, ib, f0b, f1b, cb, cnt_sh, sem0, sem1):
        c = lax.axis_index("c")
        s = lax.axis_index("s")
        bl = s % 8
        b = c * 8 + bl
        role = s // 8
        rolef = role.astype(jnp.float32)
        one_m_r = 1.0 - rolef
        ch0 = role * 2
        ch1 = 1 + role

        z = jnp.zeros((16,), jnp.float32)

        def zbody(i, _):
            base = i * 128
            for u in range(8):
                g0[pl.ds(base + u * 16, 16)] = z
                g1[pl.ds(base + u * 16, 16)] = z
            return 0

        lax.fori_loop(0, V // 128, zbody, 0)

        sems = [sem0, sem1]

        def start(kk, slot):
            r0 = kk * ROWS
            return [
                pltpu.async_copy(idx_hbm.at[b, pl.ds(r0, ROWS), :], ib.at[slot], sems[slot]),
                pltpu.async_copy(pc_hbm.at[b, ch0, pl.ds(r0, ROWS), :], f0b.at[slot], sems[slot]),
                pltpu.async_copy(pc_hbm.at[b, ch1, pl.ds(r0, ROWS), :], f1b.at[slot], sems[slot]),
            ]

        handles = [start(0, 0), start(1, 1)]
        for kk in range(NCHUNK):
            slot = kk & 1
            for h in handles[slot]:
                h.wait()

            def sbody(r, _):
                for u in range(8):
                    sl = pl.ds(u * 16, 16)
                    iv = ib[slot, r, sl]
                    v0 = f0b[slot, r, sl]
                    v1 = f1b[slot, r, sl]
                    v1 = v1 * one_m_r + rolef
                    plsc.addupdate_scatter(g0, [iv], v0)
                    plsc.addupdate_scatter(g1, [iv], v1)
                return 0

            lax.fori_loop(0, ROWS, sbody, 0)
            if kk + 2 < NCHUNK:
                handles[slot] = start(kk + 2, slot)

        @pl.when(role == 1)
        def _():
            pltpu.sync_copy(g1, cnt_sh.at[bl])

        plsc.subcore_barrier()

        def norm_block(kk, cnt_chunk_ref, targets):
            def mb(r, _):
                for u in range(8):
                    gsl = pl.ds(kk * CH + r * 128 + u * 16, 16)
                    cnt = cnt_chunk_ref[pl.ds(r * 128 + u * 16, 16)]
                    pos = cnt > 0.0
                    rec = jnp.where(pos, 1.0 / jnp.where(pos, cnt, 1.0), 0.0)
                    for g in targets:
                        g[gsl] = g[gsl] * rec
                return 0

            lax.fori_loop(0, ROWS, mb, 0)

        @pl.when(role == 0)
        def _():
            for kk in range(V // CH):
                pltpu.sync_copy(cnt_sh.at[bl, pl.ds(kk * CH, CH)], cb)
                norm_block(kk, cb, [g0, g1])
            pltpu.sync_copy(g0, out_hbm.at[b, 0])
            pltpu.sync_copy(g1, out_hbm.at[b, 1])

        @pl.when(role == 1)
        def _():
            def mb(r, _):
                for u in range(8):
                    sl = pl.ds(r * 128 + u * 16, 16)
                    cnt = g1[sl]
                    pos = cnt > 0.0
                    rec = jnp.where(pos, 1.0 / jnp.where(pos, cnt, 1.0), 0.0)
                    g0[sl] = g0[sl] * rec
                return 0

            lax.fori_loop(0, V // 128, mb, 0)
            pltpu.sync_copy(g0, out_hbm.at[b, 2])

    return k(pc4, idx4)


def _mm_body(x_ref, w_ref, b_ref, o_ref):
    o_ref[...] = (
        jnp.dot(x_ref[...], w_ref[...], preferred_element_type=jnp.float32)
        + b_ref[...]
    )


def _patch_matmul(pat, wt, bias):
    rows = B * G * G * G
    blk = 1024
    return pl.pallas_call(
        _mm_body,
        grid=(rows // blk,),
        in_specs=[
            pl.BlockSpec((blk, C * P * P * P), lambda i: (i, 0)),
            pl.BlockSpec((C * P * P * P, HIDDEN), lambda i: (0, 0)),
            pl.BlockSpec((1, HIDDEN), lambda i: (0, 0)),
        ],
        out_specs=pl.BlockSpec((blk, HIDDEN), lambda i: (i, 0)),
        out_shape=jax.ShapeDtypeStruct((rows, HIDDEN), jnp.float32),
    )(pat, wt, bias)


def kernel(point_cloud, W, b):
    pc4 = point_cloud.reshape(B, C, SUBL, LANE)
    idx4 = _voxel_indices(pc4)
    avg = _sc_voxelize(pc4, idx4)
    pat = (
        avg.reshape(B, C, G, P, G, P, G, P)
        .transpose(0, 2, 4, 6, 1, 3, 5, 7)
        .reshape(B * G * G * G, C * P * P * P)
    )
    wt = W.reshape(HIDDEN, C * P * P * P).T
    tokens = _patch_matmul(pat, wt, b.reshape(1, HIDDEN))
    return tokens.reshape(B, G * G * G, HIDDEN)
